# baseline (speedup 1.0000x reference)
import jax
import jax.numpy as jnp
from jax.experimental import pallas as pl
from jax.experimental.pallas import tpu as pltpu

_NC = 80
_MASK = 32
_MAXDET = 300
_IOU_T = 0.45
_CONF_T = 0.25
_NEG = -1e9
_N = 20000
_LANES = 128
_ROWS = 160
_NPAD = _ROWS * _LANES


def _nms_body(pt_ref, rows_ref, ob_ref, oc_ref, os_ref, om_ref,
              s_scr, y1_scr, x1_scr, y2_scr, x2_scr, ar_scr):
    obj = pt_ref[4]
    m = pt_ref[5] * obj
    for k in range(1, _NC):
        m = jnp.maximum(m, pt_ref[5 + k] * obj)
    s = jnp.where(obj > _CONF_T, m, _NEG)

    xc = pt_ref[0]
    yc = pt_ref[1]
    w2 = pt_ref[2] * 0.5
    h2 = pt_ref[3] * 0.5
    y1 = yc - h2
    x1 = xc - w2
    y2 = yc + h2
    x2 = xc + w2
    s_scr[...] = s
    y1_scr[...] = y1
    x1_scr[...] = x1
    y2_scr[...] = y2
    x2_scr[...] = x2
    ar_scr[...] = (y2 - y1) * (x2 - x1)

    iota = (jax.lax.broadcasted_iota(jnp.int32, (_ROWS, _LANES), 0) * _LANES
            + jax.lax.broadcasted_iota(jnp.int32, (_ROWS, _LANES), 1))

    def body(i, _):
        s = s_scr[...]
        best = jnp.max(s)
        idx = jnp.min(jnp.where(s == best, iota, _NPAD))
        valid = best > _NEG * 0.5

        row = rows_ref[pl.ds(idx, 1), :]
        bx = row[:, 0:1]
        by = row[:, 1:2]
        bw2 = row[:, 2:3] * 0.5
        bh2 = row[:, 3:4] * 0.5
        by1 = by - bh2
        bx1 = bx - bw2
        by2 = by + bh2
        bx2 = bx + bw2

        yy1 = jnp.maximum(y1_scr[...], by1)
        xx1 = jnp.maximum(x1_scr[...], bx1)
        yy2 = jnp.minimum(y2_scr[...], by2)
        xx2 = jnp.minimum(x2_scr[...], bx2)
        inter = (jnp.clip(yy2 - yy1, 0.0) * jnp.clip(xx2 - xx1, 0.0))
        barea = (by2 - by1) * (bx2 - bx1)
        union = ar_scr[...] + barea - inter + 1e-9
        s_new = jnp.where(inter > _IOU_T * union, _NEG, s)
        s_scr[...] = s_new

        bboxes = jnp.concatenate([by1, bx1, by2, bx2], axis=1)
        crow = row[:, 5:5 + _NC] * row[:, 4:5]
        cmax = jnp.max(crow, axis=1, keepdims=True)
        c_iota = jax.lax.broadcasted_iota(jnp.int32, (1, _NC), 1)
        cidx = jnp.min(jnp.where(crow == cmax, c_iota, _NC),
                       axis=1, keepdims=True).astype(jnp.float32)
        mrow = row[:, 5 + _NC:]

        ob_ref[pl.ds(i, 1), :] = jnp.where(valid, bboxes, 0.0)
        oc_ref[pl.ds(i, 1), :] = jnp.where(valid, cidx, 0.0)
        os_ref[pl.ds(i, 1), :] = jnp.where(valid, best, 0.0).reshape(1, 1)
        om_ref[pl.ds(i, 1), :] = jnp.where(valid, mrow, 0.0)
        return 0

    jax.lax.fori_loop(0, _MAXDET, body, 0)


@jax.jit
def kernel(predictions):
    p = predictions.reshape(_N, 5 + _NC + _MASK)
    pp = jnp.pad(p, ((0, _NPAD - _N), (0, 0)))
    pt = pp.reshape(_ROWS, _LANES, 5 + _NC + _MASK).transpose(2, 0, 1)

    out_shapes = (
        jax.ShapeDtypeStruct((_MAXDET, 4), jnp.float32),
        jax.ShapeDtypeStruct((_MAXDET, 1), jnp.float32),
        jax.ShapeDtypeStruct((_MAXDET, 1), jnp.float32),
        jax.ShapeDtypeStruct((_MAXDET, _MASK), jnp.float32),
    )
    boxes, classes, scores, masks = pl.pallas_call(
        _nms_body,
        out_shape=out_shapes,
        scratch_shapes=[pltpu.VMEM((_ROWS, _LANES), jnp.float32)
                        for _ in range(6)],
    )(pt, p)
    return (boxes[None],
            classes.reshape(1, _MAXDET),
            scores.reshape(1, _MAXDET),
            masks[None])


# speedup vs baseline: 1.0806x; 1.0806x over previous
import jax
import jax.numpy as jnp
from jax.experimental import pallas as pl
from jax.experimental.pallas import tpu as pltpu

_NC = 80
_MASK = 32
_MAXDET = 300
_IOU_T = 0.45
_CONF_T = 0.25
_NEG = -1e9
_N = 20000
_LANES = 128
_ROWS = 160
_NPAD = _ROWS * _LANES


def _nms_body(pt_ref, rows_ref, ob_ref, oc_ref, os_ref, om_ref,
              s_scr, y1_scr, x1_scr, y2_scr, x2_scr, ar_scr, key_scr):
    obj = pt_ref[4]
    m = pt_ref[5] * obj
    ci = jnp.zeros((_ROWS, _LANES), jnp.int32)
    for k in range(1, _NC):
        v = pt_ref[5 + k] * obj
        upd = v > m
        ci = jnp.where(upd, k, ci)
        m = jnp.maximum(m, v)
    s = jnp.where(obj > _CONF_T, m, _NEG)

    iota = (jax.lax.broadcasted_iota(jnp.int32, (_ROWS, _LANES), 0) * _LANES
            + jax.lax.broadcasted_iota(jnp.int32, (_ROWS, _LANES), 1))
    key_scr[...] = iota * 128 + ci

    xc = pt_ref[0]
    yc = pt_ref[1]
    w2 = pt_ref[2] * 0.5
    h2 = pt_ref[3] * 0.5
    y1 = yc - h2
    x1 = xc - w2
    y2 = yc + h2
    x2 = xc + w2
    s_scr[...] = s
    y1_scr[...] = y1
    x1_scr[...] = x1
    y2_scr[...] = y2
    x2_scr[...] = x2
    ar_scr[...] = (y2 - y1) * (x2 - x1)

    def body(i, _):
        s = s_scr[...]
        best = jnp.max(s)
        key = jnp.min(jnp.where(s == best, key_scr[...], _NPAD * 128))
        idx = key >> 7
        cls = key & 127
        valid = best > _NEG * 0.5

        row = rows_ref[pl.ds(idx, 1), :]
        bx = row[:, 0:1]
        by = row[:, 1:2]
        bw2 = row[:, 2:3] * 0.5
        bh2 = row[:, 3:4] * 0.5
        by1 = by - bh2
        bx1 = bx - bw2
        by2 = by + bh2
        bx2 = bx + bw2

        yy1 = jnp.maximum(y1_scr[...], by1)
        xx1 = jnp.maximum(x1_scr[...], bx1)
        yy2 = jnp.minimum(y2_scr[...], by2)
        xx2 = jnp.minimum(x2_scr[...], bx2)
        inter = (jnp.clip(yy2 - yy1, 0.0) * jnp.clip(xx2 - xx1, 0.0))
        barea = (by2 - by1) * (bx2 - bx1)
        union = ar_scr[...] + barea - inter + 1e-9
        s_scr[...] = jnp.where(inter > _IOU_T * union, _NEG, s)

        bboxes = jnp.concatenate([by1, bx1, by2, bx2], axis=1)
        ob_ref[pl.ds(i, 1), :] = jnp.where(valid, bboxes, 0.0)
        oc_ref[pl.ds(i, 1), :] = jnp.where(
            valid, cls.astype(jnp.float32), 0.0).reshape(1, 1)
        os_ref[pl.ds(i, 1), :] = jnp.where(valid, best, 0.0).reshape(1, 1)
        om_ref[pl.ds(i, 1), :] = jnp.where(valid, row[:, 5 + _NC:], 0.0)
        return 0

    jax.lax.fori_loop(0, _MAXDET, body, 0)


@jax.jit
def kernel(predictions):
    p = predictions.reshape(_N, 5 + _NC + _MASK)
    pp = jnp.pad(p, ((0, _NPAD - _N), (0, 0)))
    pt = pp.reshape(_ROWS, _LANES, 5 + _NC + _MASK).transpose(2, 0, 1)

    out_shapes = (
        jax.ShapeDtypeStruct((_MAXDET, 4), jnp.float32),
        jax.ShapeDtypeStruct((_MAXDET, 1), jnp.float32),
        jax.ShapeDtypeStruct((_MAXDET, 1), jnp.float32),
        jax.ShapeDtypeStruct((_MAXDET, _MASK), jnp.float32),
    )
    boxes, classes, scores, masks = pl.pallas_call(
        _nms_body,
        out_shape=out_shapes,
        scratch_shapes=([pltpu.VMEM((_ROWS, _LANES), jnp.float32)
                         for _ in range(6)]
                        + [pltpu.VMEM((_ROWS, _LANES), jnp.int32)]),
    )(pt, p)
    return (boxes[None],
            classes.reshape(1, _MAXDET),
            scores.reshape(1, _MAXDET),
            masks[None])
